# baseline (speedup 1.0000x reference)
import functools

import jax
import jax.numpy as jnp
from jax import lax
from jax.experimental import pallas as pl
from jax.experimental.pallas import tpu as pltpu
from jax.experimental.pallas import tpu_sc as plsc

_E = 64
_H = 2048
_LANES = 16
_EUNROLL = 8


def _logits_body(hs_ref, w_ref, out_ref):
    out_ref[...] = lax.dot_general(
        w_ref[...], hs_ref[...],
        dimension_numbers=(((1,), (1,)), ((), ())),
        preferred_element_type=jnp.float32,
    )


def _make_logits_t(hs2d, weight, bt):
    t = hs2d.shape[0]
    return pl.pallas_call(
        _logits_body,
        grid=(t // bt,),
        in_specs=[
            pl.BlockSpec((bt, _H), lambda i: (i, 0)),
            pl.BlockSpec((_E, _H), lambda i: (0, 0)),
        ],
        out_specs=pl.BlockSpec((_E, bt), lambda i: (0, i)),
        out_shape=jax.ShapeDtypeStruct((_E, t), jnp.float32),
    )(hs2d, weight)


def _route_body(tpw, lg_hbm, idx_hbm, w_hbm, lbuf, ibuf, wbuf,
                sem0, sem1, sem2, sem3):
    nc = 2
    wid = lax.axis_index("s") * nc + lax.axis_index("c")
    base = wid * tpw
    qw = tpw // 4
    cps = [
        pltpu.async_copy(
            lg_hbm.at[:, pl.ds(base + q * qw, qw)],
            lbuf.at[:, pl.ds(q * qw, qw)], sem)
        for q, sem in enumerate((sem0, sem1, sem2, sem3))
    ]

    nchain = 4
    cl = _E // nchain

    def merge(a1, ai1, a2, ai2, b1, bi1, b2, bi2):
        sel = a1 >= b1
        m1 = jnp.where(sel, a1, b1)
        i1 = jnp.where(sel, ai1, bi1)
        u = jnp.where(sel, a2, a1)
        ui = jnp.where(sel, ai2, ai1)
        w = jnp.where(sel, b1, b2)
        wi = jnp.where(sel, bi1, bi2)
        s2 = u >= w
        m2 = jnp.where(s2, u, w)
        i2 = jnp.where(s2, ui, wi)
        return m1, i1, m2, i2

    def chunk(ci, carry):
        off = ci * _LANES
        neg = jnp.full((_LANES,), -3.0e38, jnp.float32)
        zero = jnp.zeros((_LANES,), jnp.int32)
        sl = pl.ds(off, _LANES)
        st = [[neg, zero, neg, zero] for _ in range(nchain)]
        for j in range(cl):
            ev = zero + j
            vs = [lbuf[c * cl + j, sl] for c in range(nchain)]
            for c in range(nchain):
                m1, i1, m2, i2 = st[c]
                v = vs[c]
                gt1 = v > m1
                gt2 = v > m2
                i2 = jnp.where(gt1, i1, jnp.where(gt2, ev, i2))
                m2 = jnp.minimum(jnp.maximum(v, m2), m1)
                i1 = jnp.where(gt1, ev, i1)
                m1 = jnp.maximum(v, m1)
                st[c] = [m1, i1, m2, i2]
        for c in range(nchain):
            st[c][1] = st[c][1] + (c * cl)
            st[c][3] = st[c][3] + (c * cl)
        ab = merge(*st[0], *st[1])
        cd = merge(*st[2], *st[3])
        m1, i1, m2, i2 = merge(*ab, *cd)
        e2 = jnp.exp(m2 - m1)
        denom = 1.0 + e2
        sl = pl.ds(off, _LANES)
        ibuf[0, sl] = i1
        ibuf[1, sl] = i2
        wbuf[0, sl] = 1.0 / denom
        wbuf[1, sl] = e2 / denom
        return carry

    cpq = tpw // _LANES // 4
    for q in range(4):
        cps[q].wait()
        lax.fori_loop(q * cpq, (q + 1) * cpq, chunk, 0)
    pltpu.sync_copy(ibuf, idx_hbm.at[:, pl.ds(base, tpw)])
    pltpu.sync_copy(wbuf, w_hbm.at[:, pl.ds(base, tpw)])


def _make_route(logits_t):
    t = logits_t.shape[1]
    nw = 32
    tpw = t // nw
    mesh = plsc.VectorSubcoreMesh(core_axis_name="c", subcore_axis_name="s")
    return pl.kernel(
        functools.partial(_route_body, tpw),
        out_type=[
            jax.ShapeDtypeStruct((2, t), jnp.int32),
            jax.ShapeDtypeStruct((2, t), jnp.float32),
        ],
        mesh=mesh,
        scratch_types=[
            pltpu.VMEM((_E, tpw), jnp.float32),
            pltpu.VMEM((2, tpw), jnp.int32),
            pltpu.VMEM((2, tpw), jnp.float32),
            pltpu.SemaphoreType.DMA,
            pltpu.SemaphoreType.DMA,
            pltpu.SemaphoreType.DMA,
            pltpu.SemaphoreType.DMA,
        ],
    )(logits_t)


def kernel(hidden_states, weight):
    bsz, seq_len, d = hidden_states.shape
    t = bsz * seq_len
    hs2d = hidden_states.reshape(t, d)
    logits_t = _make_logits_t(hs2d, weight, bt=1024)
    idx_t, w_t = _make_route(logits_t)
    topk_idx = idx_t.T
    topk_weight = w_t.T
    row_idx = jnp.arange(t * 2, dtype=jnp.int32).reshape(2, t).T
    return (topk_idx, topk_weight, None, row_idx)


# speedup vs baseline: 1.0485x; 1.0485x over previous
import functools

import jax
import jax.numpy as jnp
from jax import lax
from jax.experimental import pallas as pl
from jax.experimental.pallas import tpu as pltpu
from jax.experimental.pallas import tpu_sc as plsc

_E = 64
_H = 2048
_LANES = 16
_EUNROLL = 8


def _logits_body(hs_ref, w_ref, out_ref):
    out_ref[...] = lax.dot_general(
        w_ref[...], hs_ref[...],
        dimension_numbers=(((1,), (1,)), ((), ())),
        preferred_element_type=jnp.float32,
    )


def _make_logits_t(hs2d, weight, bt):
    t = hs2d.shape[0]
    return pl.pallas_call(
        _logits_body,
        grid=(t // bt,),
        in_specs=[
            pl.BlockSpec((bt, _H), lambda i: (i, 0)),
            pl.BlockSpec((_E, _H), lambda i: (0, 0)),
        ],
        out_specs=pl.BlockSpec((_E, bt), lambda i: (0, i)),
        out_shape=jax.ShapeDtypeStruct((_E, t), jnp.float32),
    )(hs2d, weight)


def _route_body(tpw, lg_hbm, idx_hbm, w_hbm, lbuf, ibuf, wbuf, sem0, sem1):
    nc = 2
    wid = lax.axis_index("s") * nc + lax.axis_index("c")
    base = wid * tpw
    qw = tpw // 2
    cps = [
        pltpu.async_copy(
            lg_hbm.at[:, pl.ds(base + q * qw, qw)],
            lbuf.at[:, pl.ds(q * qw, qw)], sem)
        for q, sem in enumerate((sem0, sem1))
    ]

    nchain = 4
    cl = _E // nchain

    def merge(a1, ai1, a2, ai2, b1, bi1, b2, bi2):
        sel = a1 >= b1
        m1 = jnp.where(sel, a1, b1)
        i1 = jnp.where(sel, ai1, bi1)
        u = jnp.where(sel, a2, a1)
        ui = jnp.where(sel, ai2, ai1)
        w = jnp.where(sel, b1, b2)
        wi = jnp.where(sel, bi1, bi2)
        s2 = u >= w
        m2 = jnp.where(s2, u, w)
        i2 = jnp.where(s2, ui, wi)
        return m1, i1, m2, i2

    def chunk(ci, carry):
        off = ci * _LANES
        neg = jnp.full((_LANES,), -3.0e38, jnp.float32)
        zero = jnp.zeros((_LANES,), jnp.int32)
        sl = pl.ds(off, _LANES)
        st = [[neg, zero, neg, zero] for _ in range(nchain)]
        for j in range(cl):
            ev = zero + j
            vs = [lbuf[c * cl + j, sl] for c in range(nchain)]
            for c in range(nchain):
                m1, i1, m2, i2 = st[c]
                v = vs[c]
                gt1 = v > m1
                gt2 = v > m2
                i2 = jnp.where(gt1, i1, jnp.where(gt2, ev, i2))
                m2 = jnp.minimum(jnp.maximum(v, m2), m1)
                i1 = jnp.where(gt1, ev, i1)
                m1 = jnp.maximum(v, m1)
                st[c] = [m1, i1, m2, i2]
        for c in range(nchain):
            st[c][1] = st[c][1] + (c * cl)
            st[c][3] = st[c][3] + (c * cl)
        ab = merge(*st[0], *st[1])
        cd = merge(*st[2], *st[3])
        m1, i1, m2, i2 = merge(*ab, *cd)
        e2 = jnp.exp(m2 - m1)
        denom = 1.0 + e2
        sl = pl.ds(off, _LANES)
        ibuf[0, sl] = i1
        ibuf[1, sl] = i2
        wbuf[0, sl] = 1.0 / denom
        wbuf[1, sl] = e2 / denom
        return carry

    cpq = tpw // _LANES // 2
    for q in range(2):
        cps[q].wait()
        lax.fori_loop(q * cpq, (q + 1) * cpq, chunk, 0)
    pltpu.sync_copy(ibuf, idx_hbm.at[:, pl.ds(base, tpw)])
    pltpu.sync_copy(wbuf, w_hbm.at[:, pl.ds(base, tpw)])


def _make_route(logits_t):
    t = logits_t.shape[1]
    nw = 32
    tpw = t // nw
    mesh = plsc.VectorSubcoreMesh(core_axis_name="c", subcore_axis_name="s")
    return pl.kernel(
        functools.partial(_route_body, tpw),
        out_type=[
            jax.ShapeDtypeStruct((2, t), jnp.int32),
            jax.ShapeDtypeStruct((2, t), jnp.float32),
        ],
        mesh=mesh,
        scratch_types=[
            pltpu.VMEM((_E, tpw), jnp.float32),
            pltpu.VMEM((2, tpw), jnp.int32),
            pltpu.VMEM((2, tpw), jnp.float32),
            pltpu.SemaphoreType.DMA,
            pltpu.SemaphoreType.DMA,
        ],
    )(logits_t)


def kernel(hidden_states, weight):
    bsz, seq_len, d = hidden_states.shape
    t = bsz * seq_len
    hs2d = hidden_states.reshape(t, d)
    logits_t = _make_logits_t(hs2d, weight, bt=1024)
    idx_t, w_t = _make_route(logits_t)
    topk_idx = idx_t.T
    topk_weight = w_t.T
    row_idx = jnp.arange(t * 2, dtype=jnp.int32).reshape(2, t).T
    return (topk_idx, topk_weight, None, row_idx)


# speedup vs baseline: 1.0509x; 1.0022x over previous
import functools

import jax
import jax.numpy as jnp
from jax import lax
from jax.experimental import pallas as pl
from jax.experimental.pallas import tpu as pltpu
from jax.experimental.pallas import tpu_sc as plsc

_E = 64
_H = 2048
_LANES = 16
_EUNROLL = 8


def _logits_body(spw, hs_ref, w_ref, out_ref):
    for s in range(out_ref.shape[0]):
        out_ref[s] = lax.dot_general(
            w_ref[...], hs_ref[pl.ds(s * spw, spw), :],
            dimension_numbers=(((1,), (1,)), ((), ())),
            preferred_element_type=jnp.float32,
        )


def _make_logits_t(hs2d, weight, bt, spw):
    t = hs2d.shape[0]
    return pl.pallas_call(
        functools.partial(_logits_body, spw),
        grid=(t // bt,),
        in_specs=[
            pl.BlockSpec((bt, _H), lambda i: (i, 0)),
            pl.BlockSpec((_E, _H), lambda i: (0, 0)),
        ],
        out_specs=pl.BlockSpec((bt // spw, _E, spw), lambda i: (i, 0, 0)),
        out_shape=jax.ShapeDtypeStruct((t // spw, _E, spw), jnp.float32),
    )(hs2d, weight)


def _route_body(tpw, lg_hbm, idx_hbm, w_hbm, lbuf, ibuf, wbuf):
    nc = 2
    wid = lax.axis_index("s") * nc + lax.axis_index("c")
    base = wid * tpw
    pltpu.sync_copy(lg_hbm.at[wid], lbuf)

    nchain = 4
    cl = _E // nchain

    def merge(a1, ai1, a2, ai2, b1, bi1, b2, bi2):
        sel = a1 >= b1
        m1 = jnp.where(sel, a1, b1)
        i1 = jnp.where(sel, ai1, bi1)
        u = jnp.where(sel, a2, a1)
        ui = jnp.where(sel, ai2, ai1)
        w = jnp.where(sel, b1, b2)
        wi = jnp.where(sel, bi1, bi2)
        s2 = u >= w
        m2 = jnp.where(s2, u, w)
        i2 = jnp.where(s2, ui, wi)
        return m1, i1, m2, i2

    def chunk(ci, carry):
        off = ci * _LANES
        neg = jnp.full((_LANES,), -3.0e38, jnp.float32)
        zero = jnp.zeros((_LANES,), jnp.int32)
        sl = pl.ds(off, _LANES)
        st = [[neg, zero, neg, zero] for _ in range(nchain)]
        for j in range(cl):
            ev = zero + j
            vs = [lbuf[c * cl + j, sl] for c in range(nchain)]
            for c in range(nchain):
                m1, i1, m2, i2 = st[c]
                v = vs[c]
                gt1 = v > m1
                gt2 = v > m2
                i2 = jnp.where(gt1, i1, jnp.where(gt2, ev, i2))
                m2 = jnp.minimum(jnp.maximum(v, m2), m1)
                i1 = jnp.where(gt1, ev, i1)
                m1 = jnp.maximum(v, m1)
                st[c] = [m1, i1, m2, i2]
        for c in range(nchain):
            st[c][1] = st[c][1] + (c * cl)
            st[c][3] = st[c][3] + (c * cl)
        ab = merge(*st[0], *st[1])
        cd = merge(*st[2], *st[3])
        m1, i1, m2, i2 = merge(*ab, *cd)
        e2 = jnp.exp(m2 - m1)
        denom = 1.0 + e2
        sl = pl.ds(off, _LANES)
        ibuf[0, sl] = i1
        ibuf[1, sl] = i2
        wbuf[0, sl] = 1.0 / denom
        wbuf[1, sl] = e2 / denom
        return carry

    lax.fori_loop(0, tpw // _LANES, chunk, 0)
    pltpu.sync_copy(ibuf, idx_hbm.at[:, pl.ds(base, tpw)])
    pltpu.sync_copy(wbuf, w_hbm.at[:, pl.ds(base, tpw)])


def _make_route(logits_t):
    nw, _, tpw = logits_t.shape
    t = nw * tpw
    mesh = plsc.VectorSubcoreMesh(core_axis_name="c", subcore_axis_name="s")
    return pl.kernel(
        functools.partial(_route_body, tpw),
        out_type=[
            jax.ShapeDtypeStruct((2, t), jnp.int32),
            jax.ShapeDtypeStruct((2, t), jnp.float32),
        ],
        mesh=mesh,
        scratch_types=[
            pltpu.VMEM((_E, tpw), jnp.float32),
            pltpu.VMEM((2, tpw), jnp.int32),
            pltpu.VMEM((2, tpw), jnp.float32),
        ],
    )(logits_t)


def kernel(hidden_states, weight):
    bsz, seq_len, d = hidden_states.shape
    t = bsz * seq_len
    hs2d = hidden_states.reshape(t, d)
    logits_t = _make_logits_t(hs2d, weight, bt=1024, spw=t // 32)
    idx_t, w_t = _make_route(logits_t)
    topk_idx = idx_t.T
    topk_weight = w_t.T
    row_idx = jnp.arange(t * 2, dtype=jnp.int32).reshape(2, t).T
    return (topk_idx, topk_weight, None, row_idx)


# speedup vs baseline: 1.0523x; 1.0014x over previous
import functools

import jax
import jax.numpy as jnp
from jax import lax
from jax.experimental import pallas as pl
from jax.experimental.pallas import tpu as pltpu
from jax.experimental.pallas import tpu_sc as plsc

_E = 64
_H = 2048
_LANES = 16
_EUNROLL = 8


def _logits_body(spw, hs_ref, w_ref, out_ref):
    for s in range(out_ref.shape[0]):
        out_ref[s] = lax.dot_general(
            w_ref[...], hs_ref[pl.ds(s * spw, spw), :],
            dimension_numbers=(((1,), (1,)), ((), ())),
            preferred_element_type=jnp.float32,
        )


def _make_logits_t(hs2d, weight, bt, spw):
    t = hs2d.shape[0]
    return pl.pallas_call(
        functools.partial(_logits_body, spw),
        grid=(t // bt,),
        in_specs=[
            pl.BlockSpec((bt, _H), lambda i: (i, 0)),
            pl.BlockSpec((_E, _H), lambda i: (0, 0)),
        ],
        out_specs=pl.BlockSpec((bt // spw, _E, spw), lambda i: (i, 0, 0)),
        out_shape=jax.ShapeDtypeStruct((t // spw, _E, spw), jnp.float32),
    )(hs2d, weight)


def _route_body(tpw, lg_hbm, idx_hbm, w_hbm, lbuf, ibuf, wbuf, pm, pi,
                sem0, sem1):
    nc = 2
    wid = lax.axis_index("s") * nc + lax.axis_index("c")
    base = wid * tpw
    he = _E // 2
    cp0 = pltpu.async_copy(lg_hbm.at[wid, pl.ds(0, he)],
                           lbuf.at[pl.ds(0, he)], sem0)
    cp1 = pltpu.async_copy(lg_hbm.at[wid, pl.ds(he, he)],
                           lbuf.at[pl.ds(he, he)], sem1)

    nchain = 4
    cl = _E // nchain

    def merge(a1, ai1, a2, ai2, b1, bi1, b2, bi2):
        sel = a1 >= b1
        m1 = jnp.where(sel, a1, b1)
        i1 = jnp.where(sel, ai1, bi1)
        u = jnp.where(sel, a2, a1)
        ui = jnp.where(sel, ai2, ai1)
        w = jnp.where(sel, b1, b2)
        wi = jnp.where(sel, bi1, bi2)
        s2 = u >= w
        m2 = jnp.where(s2, u, w)
        i2 = jnp.where(s2, ui, wi)
        return m1, i1, m2, i2

    def run_chains(c0, off):
        neg = jnp.full((_LANES,), -3.0e38, jnp.float32)
        zero = jnp.zeros((_LANES,), jnp.int32)
        sl = pl.ds(off, _LANES)
        st = [[neg, zero, neg, zero] for _ in range(2)]
        for j in range(cl):
            ev = zero + j
            vs = [lbuf[(c0 + c) * cl + j, sl] for c in range(2)]
            for c in range(2):
                m1, i1, m2, i2 = st[c]
                v = vs[c]
                gt1 = v > m1
                gt2 = v > m2
                i2 = jnp.where(gt1, i1, jnp.where(gt2, ev, i2))
                m2 = jnp.minimum(jnp.maximum(v, m2), m1)
                i1 = jnp.where(gt1, ev, i1)
                m1 = jnp.maximum(v, m1)
                st[c] = [m1, i1, m2, i2]
        for c in range(2):
            st[c][1] = st[c][1] + ((c0 + c) * cl)
            st[c][3] = st[c][3] + ((c0 + c) * cl)
        return merge(*st[0], *st[1])

    def pass1(ci, carry):
        off = ci * _LANES
        sl = pl.ds(off, _LANES)
        m1, i1, m2, i2 = run_chains(0, off)
        pm[0, sl] = m1
        pm[1, sl] = m2
        pi[0, sl] = i1
        pi[1, sl] = i2
        return carry

    def pass2(ci, carry):
        off = ci * _LANES
        sl = pl.ds(off, _LANES)
        c1, j1, c2, j2 = run_chains(2, off)
        m1, i1, m2, i2 = merge(
            pm[0, sl], pi[0, sl], pm[1, sl], pi[1, sl], c1, j1, c2, j2)
        e2 = jnp.exp(m2 - m1)
        denom = 1.0 + e2
        ibuf[0, sl] = i1
        ibuf[1, sl] = i2
        wbuf[0, sl] = 1.0 / denom
        wbuf[1, sl] = e2 / denom
        return carry

    cp0.wait()
    lax.fori_loop(0, tpw // _LANES, pass1, 0)
    cp1.wait()
    lax.fori_loop(0, tpw // _LANES, pass2, 0)
    pltpu.sync_copy(ibuf, idx_hbm.at[:, pl.ds(base, tpw)])
    pltpu.sync_copy(wbuf, w_hbm.at[:, pl.ds(base, tpw)])


def _make_route(logits_t):
    nw, _, tpw = logits_t.shape
    t = nw * tpw
    mesh = plsc.VectorSubcoreMesh(core_axis_name="c", subcore_axis_name="s")
    return pl.kernel(
        functools.partial(_route_body, tpw),
        out_type=[
            jax.ShapeDtypeStruct((2, t), jnp.int32),
            jax.ShapeDtypeStruct((2, t), jnp.float32),
        ],
        mesh=mesh,
        scratch_types=[
            pltpu.VMEM((_E, tpw), jnp.float32),
            pltpu.VMEM((2, tpw), jnp.int32),
            pltpu.VMEM((2, tpw), jnp.float32),
            pltpu.VMEM((2, tpw), jnp.float32),
            pltpu.VMEM((2, tpw), jnp.int32),
            pltpu.SemaphoreType.DMA,
            pltpu.SemaphoreType.DMA,
        ],
    )(logits_t)


def kernel(hidden_states, weight):
    bsz, seq_len, d = hidden_states.shape
    t = bsz * seq_len
    hs2d = hidden_states.reshape(t, d)
    logits_t = _make_logits_t(hs2d, weight, bt=1024, spw=t // 32)
    idx_t, w_t = _make_route(logits_t)
    topk_idx = idx_t.T
    topk_weight = w_t.T
    row_idx = jnp.arange(t * 2, dtype=jnp.int32).reshape(2, t).T
    return (topk_idx, topk_weight, None, row_idx)
